# Initial kernel scaffold; baseline (speedup 1.0000x reference)
import functools

import jax
import jax.numpy as jnp
from jax import lax
from jax.experimental import pallas as pl
from jax.experimental.pallas import tpu as pltpu
from jax.experimental.pallas import tpu_sc as plsc

N_NODES = 10000
N_EDGES = 320000
D = 128

NC = 2
NS = 16
NW = NC * NS

B = 128
NB = 80
E_PAD = NW * NB * B
CH = 626
N_PAD = NS * CH
DUMMY_DST = N_NODES + 8


def _mm_body(x_ref, w_ref, b_ref, o_ref):
    o_ref[...] = (
        jnp.dot(x_ref[...], w_ref[...], preferred_element_type=jnp.float32)
        + b_ref[...]
    )


def _matmul(x, W, b2d):
    m_blk = 1000
    return pl.pallas_call(
        _mm_body,
        grid=(N_NODES // m_blk,),
        in_specs=[
            pl.BlockSpec((m_blk, D), lambda i: (i, 0)),
            pl.BlockSpec((D, D), lambda i: (0, 0)),
            pl.BlockSpec((1, D), lambda i: (0, 0)),
        ],
        out_specs=pl.BlockSpec((m_blk, D), lambda i: (i, 0)),
        out_shape=jax.ShapeDtypeStruct((N_NODES, D), jnp.float32),
    )(x, W, b2d)


_sc_mesh = plsc.VectorSubcoreMesh(core_axis_name="c", subcore_axis_name="s")


@functools.partial(
    pl.kernel,
    out_type=jax.ShapeDtypeStruct((NC, N_PAD, D), jnp.float32),
    mesh=_sc_mesh,
    scratch_types=[
        pltpu.VMEM((NB, B), jnp.int32),
        pltpu.VMEM((NB, B), jnp.int32),
        pltpu.VMEM((B, D), jnp.float32),
        pltpu.VMEM_SHARED((N_PAD, D), jnp.float32),
        pltpu.SemaphoreType.DMA,
    ],
)
def _sc_push(h_hbm, srcs_hbm, dsts_hbm, zeros_hbm, out_hbm,
             src_v, dst_v, rows_v, acc, sem):
    c = lax.axis_index("c")
    s = lax.axis_index("s")
    wid = c * NS + s

    pltpu.sync_copy(srcs_hbm.at[wid], src_v)
    pltpu.sync_copy(dsts_hbm.at[wid], dst_v)
    pltpu.sync_copy(zeros_hbm.at[pl.ds(s * CH, CH)], acc.at[pl.ds(s * CH, CH)])
    plsc.subcore_barrier()

    @pl.loop(0, NB)
    def _batch(j):
        pltpu.async_copy(h_hbm.at[src_v.at[j]], rows_v, sem).wait()
        pltpu.sync_copy(rows_v, acc.at[dst_v.at[j]], add=True)

    plsc.subcore_barrier()
    pltpu.sync_copy(acc.at[pl.ds(s * CH, CH)],
                    out_hbm.at[c, pl.ds(s * CH, CH)])


def _comb_body(a_ref, b_ref, o_ref):
    o_ref[...] = jnp.maximum(a_ref[...] + b_ref[...], 0.0)


def _combine(a, b):
    m_blk = 1000
    return pl.pallas_call(
        _comb_body,
        grid=(N_NODES // m_blk,),
        in_specs=[
            pl.BlockSpec((m_blk, D), lambda i: (i, 0)),
            pl.BlockSpec((m_blk, D), lambda i: (i, 0)),
        ],
        out_specs=pl.BlockSpec((m_blk, D), lambda i: (i, 0)),
        out_shape=jax.ShapeDtypeStruct((N_NODES, D), jnp.float32),
    )(a, b)


@jax.jit
def kernel(x, edge_index, W, b):
    h = _matmul(x, W, b.reshape(1, D))

    src = edge_index[0].astype(jnp.int32)
    dst = edge_index[1].astype(jnp.int32)
    pad = E_PAD - N_EDGES
    src = jnp.concatenate([src, jnp.zeros((pad,), jnp.int32)])
    dst = jnp.concatenate([dst, jnp.full((pad,), DUMMY_DST, jnp.int32)])
    srcs = src.reshape(NW, NB, B)
    dsts = dst.reshape(NW, NB, B)
    zeros = jnp.zeros((N_PAD, D), jnp.float32)

    acc = _sc_push(h, srcs, dsts, zeros)
    return _combine(acc[0, :N_NODES], acc[1, :N_NODES])


# speedup vs baseline: 3.2130x; 3.2130x over previous
import functools

import jax
import jax.numpy as jnp
from jax import lax
from jax.experimental import pallas as pl
from jax.experimental.pallas import tpu as pltpu
from jax.experimental.pallas import tpu_sc as plsc

N_NODES = 10000
N_EDGES = 320000
D = 128

NC = 2
NS = 16
NW = NC * NS

B = 128
NB = 80
E_PAD = NW * NB * B
CH = 632
N_PAD = NS * CH
DUMMY_DST = N_NODES + 8


def _mm_body(x_ref, w_ref, b_ref, o_ref):
    o_ref[...] = (
        jnp.dot(x_ref[...], w_ref[...], preferred_element_type=jnp.float32)
        + b_ref[...]
    )


def _matmul(x, W, b2d):
    m_blk = 1000
    return pl.pallas_call(
        _mm_body,
        grid=(N_NODES // m_blk,),
        in_specs=[
            pl.BlockSpec((m_blk, D), lambda i: (i, 0)),
            pl.BlockSpec((D, D), lambda i: (0, 0)),
            pl.BlockSpec((1, D), lambda i: (0, 0)),
        ],
        out_specs=pl.BlockSpec((m_blk, D), lambda i: (i, 0)),
        out_shape=jax.ShapeDtypeStruct((N_NODES, D), jnp.float32),
    )(x, W, b2d)


_sc_mesh = plsc.VectorSubcoreMesh(core_axis_name="c", subcore_axis_name="s")


@functools.partial(
    pl.kernel,
    out_type=jax.ShapeDtypeStruct((NC, N_PAD, D), jnp.float32),
    mesh=_sc_mesh,
    scratch_types=[
        pltpu.VMEM((NB, B), jnp.int32),
        pltpu.VMEM((NB, B), jnp.int32),
        pltpu.VMEM((B, D), jnp.float32),
        pltpu.VMEM_SHARED((N_PAD, D), jnp.float32),
        pltpu.SemaphoreType.DMA,
    ],
)
def _sc_push(h_hbm, srcs_hbm, dsts_hbm, zeros_hbm, out_hbm,
             src_v, dst_v, rows_v, acc, sem):
    c = lax.axis_index("c")
    s = lax.axis_index("s")
    wid = c * NS + s

    pltpu.sync_copy(srcs_hbm.at[wid], src_v)
    pltpu.sync_copy(dsts_hbm.at[wid], dst_v)
    pltpu.sync_copy(zeros_hbm.at[pl.ds(s * CH, CH)], acc.at[pl.ds(s * CH, CH)])
    plsc.subcore_barrier()

    @pl.loop(0, NB)
    def _batch(j):
        pltpu.async_copy(h_hbm.at[src_v.at[j]], rows_v, sem).wait()
        pltpu.sync_copy(rows_v, acc.at[dst_v.at[j]], add=True)

    plsc.subcore_barrier()
    pltpu.sync_copy(acc.at[pl.ds(s * CH, CH)],
                    out_hbm.at[c, pl.ds(s * CH, CH)])


def _comb_body(a_ref, b_ref, o_ref):
    o_ref[...] = jnp.maximum(a_ref[...] + b_ref[...], 0.0)


def _combine(a, b):
    m_blk = 1000
    return pl.pallas_call(
        _comb_body,
        grid=(N_NODES // m_blk,),
        in_specs=[
            pl.BlockSpec((m_blk, D), lambda i: (i, 0)),
            pl.BlockSpec((m_blk, D), lambda i: (i, 0)),
        ],
        out_specs=pl.BlockSpec((m_blk, D), lambda i: (i, 0)),
        out_shape=jax.ShapeDtypeStruct((N_NODES, D), jnp.float32),
    )(a, b)


@jax.jit
def kernel(x, edge_index, W, b):
    h = _matmul(x, W, b.reshape(1, D))

    src = edge_index[0].astype(jnp.int32)
    dst = edge_index[1].astype(jnp.int32)
    pad = E_PAD - N_EDGES
    src = jnp.concatenate([src, jnp.zeros((pad,), jnp.int32)])
    dst = jnp.concatenate([dst, jnp.full((pad,), DUMMY_DST, jnp.int32)])
    srcs = src.reshape(NW, NB, B)
    dsts = dst.reshape(NW, NB, B)
    zeros = jnp.zeros((N_PAD, D), jnp.float32)

    acc = _sc_push(h, srcs, dsts, zeros)
    return _combine(acc[0, :N_NODES], acc[1, :N_NODES])


# speedup vs baseline: 3.5852x; 1.1158x over previous
import functools

import jax
import jax.numpy as jnp
from jax import lax
from jax.experimental import pallas as pl
from jax.experimental.pallas import tpu as pltpu
from jax.experimental.pallas import tpu_sc as plsc

N_NODES = 10000
N_EDGES = 320000
D = 128

NC = 2
NS = 16
NW = NC * NS

B = 128
NB = 80
E_PAD = NW * NB * B
CH = 632
N_PAD = NS * CH
DUMMY_DST = N_NODES + 8


def _mm_body(x_ref, w_ref, b_ref, o_ref):
    o_ref[...] = (
        jnp.dot(x_ref[...], w_ref[...], preferred_element_type=jnp.float32)
        + b_ref[...]
    )


def _matmul(x, W, b2d):
    m_blk = 1000
    return pl.pallas_call(
        _mm_body,
        grid=(N_NODES // m_blk,),
        in_specs=[
            pl.BlockSpec((m_blk, D), lambda i: (i, 0)),
            pl.BlockSpec((D, D), lambda i: (0, 0)),
            pl.BlockSpec((1, D), lambda i: (0, 0)),
        ],
        out_specs=pl.BlockSpec((m_blk, D), lambda i: (i, 0)),
        out_shape=jax.ShapeDtypeStruct((N_NODES, D), jnp.float32),
    )(x, W, b2d)


_sc_mesh = plsc.VectorSubcoreMesh(core_axis_name="c", subcore_axis_name="s")


@functools.partial(
    pl.kernel,
    out_type=jax.ShapeDtypeStruct((NC, N_PAD, D), jnp.float32),
    mesh=_sc_mesh,
    scratch_types=[
        pltpu.VMEM((NB, B), jnp.int32),
        pltpu.VMEM((2, B), jnp.int32),
        [pltpu.VMEM((B, D), jnp.float32) for _ in range(2)],
        pltpu.VMEM_SHARED((N_PAD, D), jnp.float32),
        [pltpu.SemaphoreType.DMA for _ in range(2)],
        [pltpu.SemaphoreType.DMA for _ in range(2)],
    ],
)
def _sc_push(h_hbm, srcs_hbm, dsts_hbm, zeros_hbm, out_hbm,
             src_v, dst_ring, rows, acc, rsems, dsems):
    c = lax.axis_index("c")
    s = lax.axis_index("s")
    wid = c * NS + s

    pltpu.sync_copy(srcs_hbm.at[wid], src_v)
    pltpu.sync_copy(zeros_hbm.at[pl.ds(s * CH, CH)], acc.at[pl.ds(s * CH, CH)])
    plsc.subcore_barrier()

    nbuf = 2
    for b in range(nbuf):
        pltpu.async_copy(h_hbm.at[src_v.at[b]], rows[b], rsems[b])
        pltpu.async_copy(dsts_hbm.at[wid, b], dst_ring.at[b], dsems[b])

    @pl.loop(0, NB - nbuf, step=nbuf)
    def _batch(g):
        for b in range(nbuf):
            j = g + b
            pltpu.make_async_copy(h_hbm.at[src_v.at[j]], rows[b],
                                  rsems[b]).wait()
            pltpu.make_async_copy(dsts_hbm.at[wid, j], dst_ring.at[b],
                                  dsems[b]).wait()
            pltpu.sync_copy(rows[b], acc.at[dst_ring.at[b]], add=True)
            pltpu.async_copy(h_hbm.at[src_v.at[j + nbuf]], rows[b], rsems[b])
            pltpu.async_copy(dsts_hbm.at[wid, j + nbuf], dst_ring.at[b],
                             dsems[b])

    for b in range(nbuf):
        j = NB - nbuf + b
        pltpu.make_async_copy(h_hbm.at[src_v.at[j]], rows[b], rsems[b]).wait()
        pltpu.make_async_copy(dsts_hbm.at[wid, j], dst_ring.at[b],
                              dsems[b]).wait()
        pltpu.sync_copy(rows[b], acc.at[dst_ring.at[b]], add=True)

    plsc.subcore_barrier()
    pltpu.sync_copy(acc.at[pl.ds(s * CH, CH)],
                    out_hbm.at[c, pl.ds(s * CH, CH)])


def _comb_body(a_ref, b_ref, o_ref):
    o_ref[...] = jnp.maximum(a_ref[...] + b_ref[...], 0.0)


def _combine(a, b):
    m_blk = 1000
    return pl.pallas_call(
        _comb_body,
        grid=(N_NODES // m_blk,),
        in_specs=[
            pl.BlockSpec((m_blk, D), lambda i: (i, 0)),
            pl.BlockSpec((m_blk, D), lambda i: (i, 0)),
        ],
        out_specs=pl.BlockSpec((m_blk, D), lambda i: (i, 0)),
        out_shape=jax.ShapeDtypeStruct((N_NODES, D), jnp.float32),
    )(a, b)


@jax.jit
def kernel(x, edge_index, W, b):
    h = _matmul(x, W, b.reshape(1, D))

    src = edge_index[0].astype(jnp.int32)
    dst = edge_index[1].astype(jnp.int32)
    pad = E_PAD - N_EDGES
    src = jnp.concatenate([src, jnp.zeros((pad,), jnp.int32)])
    dst = jnp.concatenate([dst, jnp.full((pad,), DUMMY_DST, jnp.int32)])
    srcs = src.reshape(NW, NB, B)
    dsts = dst.reshape(NW, NB, B)
    zeros = jnp.zeros((N_PAD, D), jnp.float32)

    acc = _sc_push(h, srcs, dsts, zeros)
    return _combine(acc[0, :N_NODES], acc[1, :N_NODES])
